# baseline (speedup 1.0000x reference)
import functools
import jax
import jax.numpy as jnp
from jax import lax
from jax.experimental import pallas as pl
from jax.experimental.pallas import tpu as pltpu
from jax.experimental.pallas import tpu_sc as plsc

B, S, D = 2, 8192, 2048
NC, NSUB = 2, 16
NW = NC * NSUB
DSTRIP = 128
ND = D // DSTRIP
ST = 128
NT = S // ST
NG = DSTRIP // 16


MW = 16


def _sc_body(x_hbm, m_hbm, o_hbm, xbuf, mbuf, obuf, is0, is1, os0, os1):
    wid = lax.axis_index("s") * NC + lax.axis_index("c")
    b = wid // ND
    d0 = (wid % ND) * DSTRIP
    strip = wid % ND
    isems = (is0, is1)
    osems = (os0, os1)

    def start_in(t, k):
        pltpu.async_copy(
            x_hbm.at[b, pl.ds(t * ST, ST), pl.ds(d0, DSTRIP)],
            xbuf.at[k], isems[k])
        pltpu.async_copy(
            m_hbm.at[b, strip, pl.ds(t * ST, ST), :],
            mbuf.at[k], isems[k])

    def wait_in(k):
        pltpu.make_async_copy(
            x_hbm.at[0, pl.ds(0, ST), pl.ds(0, DSTRIP)],
            xbuf.at[k], isems[k]).wait()
        pltpu.make_async_copy(
            m_hbm.at[0, 0, pl.ds(0, ST), :],
            mbuf.at[k], isems[k]).wait()

    def start_out(t, k):
        pltpu.async_copy(
            obuf.at[k],
            o_hbm.at[b, pl.ds(t * ST, ST), pl.ds(d0, DSTRIP)],
            osems[k])

    def wait_out(k):
        pltpu.make_async_copy(
            obuf.at[k],
            o_hbm.at[0, pl.ds(0, ST), pl.ds(0, DSTRIP)],
            osems[k]).wait()

    start_in(0, 0)
    start_in(1, 1)

    def rows(k, accs):
        def rowbody(r, accs):
            mw = mbuf[k, r, :]
            new = []
            for g in range(NG):
                mf = (mw >> g) & jnp.int32(1)
                xg = xbuf[k, r, pl.ds(g * 16, 16)]
                a = accs[g] + xg * mf.astype(jnp.float32)
                obuf[k, r, pl.ds(g * 16, 16)] = a
                new.append(a)
            return tuple(new)
        return lax.fori_loop(0, ST, rowbody, accs)

    accs = tuple(jnp.zeros((16,), jnp.float32) for _ in range(NG))

    def tile2(i2, accs):
        for k in range(2):
            t = i2 * 2 + k

            wait_in(k)

            @pl.when(i2 >= 1)
            def _():
                wait_out(k)

            accs = rows(k, accs)
            start_out(t, k)

            @pl.when(t + 2 < NT)
            def _():
                start_in(t + 2, k)
        return accs

    lax.fori_loop(0, NT // 2, tile2, accs)
    wait_out(0)
    wait_out(1)


@functools.partial(
    pl.kernel,
    out_type=jax.ShapeDtypeStruct((B, S, D), jnp.float32),
    mesh=plsc.VectorSubcoreMesh(core_axis_name="c", subcore_axis_name="s"),
    scratch_types=[
        pltpu.VMEM((2, ST, DSTRIP), jnp.float32),
        pltpu.VMEM((2, ST, MW), jnp.int32),
        pltpu.VMEM((2, ST, DSTRIP), jnp.float32),
        pltpu.SemaphoreType.DMA,
        pltpu.SemaphoreType.DMA,
        pltpu.SemaphoreType.DMA,
        pltpu.SemaphoreType.DMA,
    ],
    compiler_params=pltpu.CompilerParams(use_tc_tiling_on_sc=False),
)
def _sc_kernel(x_hbm, m_hbm, o_hbm, *rest):
    _sc_body(x_hbm, m_hbm, o_hbm, *rest)


def kernel(x, mask):
    mr = mask.reshape(B, S, ND, NG, 16).astype(jnp.int32)
    shifts = jnp.arange(NG, dtype=jnp.int32)[None, None, None, :, None]
    mp = (mr << shifts).sum(axis=3).transpose(0, 2, 1, 3)
    return _sc_kernel(x, mp)


# speedup vs baseline: 2.4424x; 2.4424x over previous
import functools
import jax
import jax.numpy as jnp
from jax import lax
from jax.experimental import pallas as pl
from jax.experimental.pallas import tpu as pltpu
from jax.experimental.pallas import tpu_sc as plsc

B, S, D = 2, 8192, 2048
NC, NSUB = 2, 16
NW = NC * NSUB
DSTRIP = 128
ND = D // DSTRIP
ST = 128
NT = S // ST
NG = DSTRIP // 16
SR = 8
PBLK = 512


def _pack_body(m_ref, o_ref):
    m = m_ref[0].astype(jnp.int32)
    m3 = m.reshape(PBLK // SR, SR, D)
    sh = lax.broadcasted_iota(jnp.int32, (1, SR, 1), 1)
    o_ref[0] = (m3 << sh).sum(axis=1)


def _pack(mask):
    grid = (B, S // PBLK)
    return pl.pallas_call(
        _pack_body,
        grid=grid,
        in_specs=[pl.BlockSpec((1, PBLK, D), lambda b, s: (b, s, 0))],
        out_specs=pl.BlockSpec((1, PBLK // SR, D), lambda b, s: (b, s, 0)),
        out_shape=jax.ShapeDtypeStruct((B, S // SR, D), jnp.int32),
        compiler_params=pltpu.CompilerParams(
            dimension_semantics=("parallel", "parallel")),
    )(mask)


def _sc_body(x_hbm, m_hbm, o_hbm, xbuf, mbuf, obuf, is0, is1, os0, os1):
    wid = lax.axis_index("s") * NC + lax.axis_index("c")
    b = wid // ND
    d0 = (wid % ND) * DSTRIP
    isems = (is0, is1)
    osems = (os0, os1)

    def start_in(t, k):
        pltpu.async_copy(
            x_hbm.at[b, pl.ds(t * ST, ST), pl.ds(d0, DSTRIP)],
            xbuf.at[k], isems[k])
        pltpu.async_copy(
            m_hbm.at[b, pl.ds(t * (ST // SR), ST // SR), pl.ds(d0, DSTRIP)],
            mbuf.at[k], isems[k])

    def wait_in(k):
        pltpu.make_async_copy(
            x_hbm.at[0, pl.ds(0, ST), pl.ds(0, DSTRIP)],
            xbuf.at[k], isems[k]).wait()
        pltpu.make_async_copy(
            m_hbm.at[0, pl.ds(0, ST // SR), pl.ds(0, DSTRIP)],
            mbuf.at[k], isems[k]).wait()

    def start_out(t, k):
        pltpu.async_copy(
            obuf.at[k],
            o_hbm.at[b, pl.ds(t * ST, ST), pl.ds(d0, DSTRIP)],
            osems[k])

    def wait_out(k):
        pltpu.make_async_copy(
            obuf.at[k],
            o_hbm.at[0, pl.ds(0, ST), pl.ds(0, DSTRIP)],
            osems[k]).wait()

    start_in(0, 0)
    start_in(1, 1)

    def rows(k, accs):
        def rowbody(r8, accs):
            wv = [mbuf[k, r8, pl.ds(gg * 16, 16)] for gg in range(NG)]
            cur = list(accs)
            for g in range(SR):
                r = r8 * SR + g
                for gg in range(NG):
                    mf = (wv[gg] >> g) & jnp.int32(1)
                    xg = xbuf[k, r, pl.ds(gg * 16, 16)]
                    a = cur[gg] + xg * mf.astype(jnp.float32)
                    obuf[k, r, pl.ds(gg * 16, 16)] = a
                    cur[gg] = a
            return tuple(cur)
        return lax.fori_loop(0, ST // SR, rowbody, accs)

    accs = tuple(jnp.zeros((16,), jnp.float32) for _ in range(NG))

    def tile2(i2, accs):
        for k in range(2):
            t = i2 * 2 + k

            wait_in(k)

            @pl.when(i2 >= 1)
            def _():
                wait_out(k)

            accs = rows(k, accs)
            start_out(t, k)

            @pl.when(t + 2 < NT)
            def _():
                start_in(t + 2, k)
        return accs

    lax.fori_loop(0, NT // 2, tile2, accs)
    wait_out(0)
    wait_out(1)


@functools.partial(
    pl.kernel,
    out_type=jax.ShapeDtypeStruct((B, S, D), jnp.float32),
    mesh=plsc.VectorSubcoreMesh(core_axis_name="c", subcore_axis_name="s"),
    scratch_types=[
        pltpu.VMEM((2, ST, DSTRIP), jnp.float32),
        pltpu.VMEM((2, ST // SR, DSTRIP), jnp.int32),
        pltpu.VMEM((2, ST, DSTRIP), jnp.float32),
        pltpu.SemaphoreType.DMA,
        pltpu.SemaphoreType.DMA,
        pltpu.SemaphoreType.DMA,
        pltpu.SemaphoreType.DMA,
    ],
)
def _sc_kernel(x_hbm, m_hbm, o_hbm, *rest):
    _sc_body(x_hbm, m_hbm, o_hbm, *rest)


def kernel(x, mask):
    return _sc_kernel(x, _pack(mask))


# speedup vs baseline: 3.0523x; 1.2497x over previous
import functools
import jax
import jax.numpy as jnp
from jax import lax
from jax.experimental import pallas as pl
from jax.experimental.pallas import tpu as pltpu
from jax.experimental.pallas import tpu_sc as plsc

B, S, D = 2, 8192, 2048
NC, NSUB = 2, 16
NW = NC * NSUB
DSTRIP = 128
ND = D // DSTRIP
ST = 128
NT = S // ST
NG = DSTRIP // 16
SR = 8
PBLK = 512


def _pack_body(m_ref, o_ref):
    m = m_ref[0].astype(jnp.int32)
    m3 = m.reshape(PBLK // SR, SR, D)
    sh = lax.broadcasted_iota(jnp.int32, (1, SR, 1), 1)
    o_ref[0] = (m3 << sh).sum(axis=1)


def _pack(mask):
    grid = (B, S // PBLK)
    return pl.pallas_call(
        _pack_body,
        grid=grid,
        in_specs=[pl.BlockSpec((1, PBLK, D), lambda b, s: (b, s, 0))],
        out_specs=pl.BlockSpec((1, PBLK // SR, D), lambda b, s: (b, s, 0)),
        out_shape=jax.ShapeDtypeStruct((B, S // SR, D), jnp.int32),
        compiler_params=pltpu.CompilerParams(
            dimension_semantics=("parallel", "parallel")),
    )(mask)


def _sc_body(x_hbm, m_hbm, o_hbm, xbuf, mbuf, obuf, is0, is1, os0, os1):
    wid = lax.axis_index("s") * NC + lax.axis_index("c")
    b = wid // ND
    d0 = (wid % ND) * DSTRIP
    isems = (is0, is1)
    osems = (os0, os1)

    def start_in(t, k):
        pltpu.async_copy(
            x_hbm.at[b, pl.ds(t * ST, ST), pl.ds(d0, DSTRIP)],
            xbuf.at[k], isems[k])
        pltpu.async_copy(
            m_hbm.at[b, pl.ds(t * (ST // SR), ST // SR), pl.ds(d0, DSTRIP)],
            mbuf.at[k], isems[k])

    def wait_in(k):
        pltpu.make_async_copy(
            x_hbm.at[0, pl.ds(0, ST), pl.ds(0, DSTRIP)],
            xbuf.at[k], isems[k]).wait()
        pltpu.make_async_copy(
            m_hbm.at[0, pl.ds(0, ST // SR), pl.ds(0, DSTRIP)],
            mbuf.at[k], isems[k]).wait()

    def start_out(t, k):
        pltpu.async_copy(
            obuf.at[k],
            o_hbm.at[b, pl.ds(t * ST, ST), pl.ds(d0, DSTRIP)],
            osems[k])

    def wait_out(k):
        pltpu.make_async_copy(
            obuf.at[k],
            o_hbm.at[0, pl.ds(0, ST), pl.ds(0, DSTRIP)],
            osems[k]).wait()

    start_in(0, 0)
    start_in(1, 1)

    def rows(k, accs):
        def rowbody(r8, accs):
            wv = [mbuf[k, r8, pl.ds(gg * 16, 16)] for gg in range(NG)]
            cur = list(accs)
            for g in range(SR):
                r = r8 * SR + g
                for gg in range(NG):
                    mf = (wv[gg] >> g) & jnp.int32(1)
                    xg = xbuf[k, r, pl.ds(gg * 16, 16)]
                    a = cur[gg] + xg * mf.astype(jnp.float32)
                    obuf[k, r, pl.ds(gg * 16, 16)] = a
                    cur[gg] = a
            return tuple(cur)
        return lax.fori_loop(0, ST // SR, rowbody, accs)

    accs = tuple(jnp.zeros((16,), jnp.float32) for _ in range(NG))

    def tile2(i2, accs):
        for k in range(2):
            t = i2 * 2 + k

            wait_in(k)

            @pl.when(i2 >= 1)
            def _():
                wait_out(k)

            accs = rows(k, accs)
            start_out(t, k)

            @pl.when(t + 2 < NT)
            def _():
                start_in(t + 2, k)
        return accs

    lax.fori_loop(0, NT // 2, tile2, accs)
    wait_out(0)
    wait_out(1)


@functools.partial(
    pl.kernel,
    out_type=jax.ShapeDtypeStruct((B, S, D), jnp.float32),
    mesh=plsc.VectorSubcoreMesh(core_axis_name="c", subcore_axis_name="s"),
    scratch_types=[
        pltpu.VMEM((2, ST, DSTRIP), jnp.float32),
        pltpu.VMEM((2, ST // SR, DSTRIP), jnp.int32),
        pltpu.VMEM((2, ST, DSTRIP), jnp.float32),
        pltpu.SemaphoreType.DMA,
        pltpu.SemaphoreType.DMA,
        pltpu.SemaphoreType.DMA,
        pltpu.SemaphoreType.DMA,
    ],
)
def _sc_kernel(x_hbm, m_hbm, o_hbm, *rest):
    _sc_body(x_hbm, m_hbm, o_hbm, *rest)


def kernel(x, mask):
    return _sc_kernel(x, _pack(mask.view(jnp.uint8)))


# speedup vs baseline: 3.1173x; 1.0213x over previous
import functools
import jax
import jax.numpy as jnp
from jax import lax
from jax.experimental import pallas as pl
from jax.experimental.pallas import tpu as pltpu
from jax.experimental.pallas import tpu_sc as plsc

B, S, D = 2, 8192, 2048
NC, NSUB = 2, 16
NW = NC * NSUB
DSTRIP = 128
ND = D // DSTRIP
ST = 128
NT = S // ST
NG = DSTRIP // 16
SR = 8
PBLK = 512


def _pack_body(m_ref, o_ref):
    m = m_ref[0].astype(jnp.int32)
    m3 = m.reshape(PBLK // SR, SR, D)
    sh = lax.broadcasted_iota(jnp.int32, (1, SR, 1), 1)
    o_ref[0] = (m3 << sh).sum(axis=1)


def _pack(mask):
    grid = (B, S // PBLK)
    return pl.pallas_call(
        _pack_body,
        grid=grid,
        in_specs=[pl.BlockSpec((1, PBLK, D), lambda b, s: (b, s, 0))],
        out_specs=pl.BlockSpec((1, PBLK // SR, D), lambda b, s: (b, s, 0)),
        out_shape=jax.ShapeDtypeStruct((B, S // SR, D), jnp.int32),
        compiler_params=pltpu.CompilerParams(
            dimension_semantics=("parallel", "parallel")),
    )(mask)


def _sc_body(x_hbm, m_hbm, o_hbm, xbuf, mbuf, obuf, is0, is1, os0, os1):
    wid = lax.axis_index("s") * NC + lax.axis_index("c")
    b = wid // ND
    d0 = (wid % ND) * DSTRIP
    isems = (is0, is1)
    osems = (os0, os1)

    def start_in(t, k):
        pltpu.async_copy(
            x_hbm.at[b, pl.ds(t * ST, ST), pl.ds(d0, DSTRIP)],
            xbuf.at[k], isems[k])
        pltpu.async_copy(
            m_hbm.at[b, pl.ds(t * (ST // SR), ST // SR), pl.ds(d0, DSTRIP)],
            mbuf.at[k], isems[k])

    def wait_in(k):
        pltpu.make_async_copy(
            x_hbm.at[0, pl.ds(0, ST), pl.ds(0, DSTRIP)],
            xbuf.at[k], isems[k]).wait()
        pltpu.make_async_copy(
            m_hbm.at[0, pl.ds(0, ST // SR), pl.ds(0, DSTRIP)],
            mbuf.at[k], isems[k]).wait()

    def start_out(t, k):
        pltpu.async_copy(
            obuf.at[k],
            o_hbm.at[b, pl.ds(t * ST, ST), pl.ds(d0, DSTRIP)],
            osems[k])

    def wait_out(k):
        pltpu.make_async_copy(
            obuf.at[k],
            o_hbm.at[0, pl.ds(0, ST), pl.ds(0, DSTRIP)],
            osems[k]).wait()

    start_in(0, 0)
    start_in(1, 1)

    def rows(k, accs):
        def rowbody(r8, accs):
            wv = [mbuf[k, r8, pl.ds(gg * 16, 16)] for gg in range(NG)]
            cur = list(accs)
            for g in range(SR):
                r = r8 * SR + g
                for gg in range(NG):
                    cond = (wv[gg] & (1 << g)) != 0
                    xg = xbuf[k, r, pl.ds(gg * 16, 16)]
                    a = cur[gg] + jnp.where(cond, xg, 0.0)
                    obuf[k, r, pl.ds(gg * 16, 16)] = a
                    cur[gg] = a
            return tuple(cur)
        return lax.fori_loop(0, ST // SR, rowbody, accs)

    accs = tuple(jnp.zeros((16,), jnp.float32) for _ in range(NG))

    def tile2(i2, accs):
        for k in range(2):
            t = i2 * 2 + k

            wait_in(k)

            @pl.when(i2 >= 1)
            def _():
                wait_out(k)

            accs = rows(k, accs)
            start_out(t, k)

            @pl.when(t + 2 < NT)
            def _():
                start_in(t + 2, k)
        return accs

    lax.fori_loop(0, NT // 2, tile2, accs)
    wait_out(0)
    wait_out(1)


@functools.partial(
    pl.kernel,
    out_type=jax.ShapeDtypeStruct((B, S, D), jnp.float32),
    mesh=plsc.VectorSubcoreMesh(core_axis_name="c", subcore_axis_name="s"),
    scratch_types=[
        pltpu.VMEM((2, ST, DSTRIP), jnp.float32),
        pltpu.VMEM((2, ST // SR, DSTRIP), jnp.int32),
        pltpu.VMEM((2, ST, DSTRIP), jnp.float32),
        pltpu.SemaphoreType.DMA,
        pltpu.SemaphoreType.DMA,
        pltpu.SemaphoreType.DMA,
        pltpu.SemaphoreType.DMA,
    ],
)
def _sc_kernel(x_hbm, m_hbm, o_hbm, *rest):
    _sc_body(x_hbm, m_hbm, o_hbm, *rest)


def kernel(x, mask):
    return _sc_kernel(x, _pack(mask.view(jnp.uint8)))


# speedup vs baseline: 3.1620x; 1.0143x over previous
import functools
import jax
import jax.numpy as jnp
from jax import lax
from jax.experimental import pallas as pl
from jax.experimental.pallas import tpu as pltpu
from jax.experimental.pallas import tpu_sc as plsc

B, S, D = 2, 8192, 2048
NC, NSUB = 2, 16
NW = NC * NSUB
DSTRIP = 128
ND = D // DSTRIP
ST = 128
NT = S // ST
NG = DSTRIP // 16
SR = 8
PBLK = 512


def _pack_body(m_ref, o_ref):
    m = m_ref[0].astype(jnp.int32)
    m3 = m.reshape(PBLK // SR, SR, D)
    sh = lax.broadcasted_iota(jnp.int32, (1, SR, 1), 1)
    o_ref[0] = (m3 << sh).sum(axis=1)


def _pack(mask8):
    grid = (B, S // PBLK)
    return pl.pallas_call(
        _pack_body,
        grid=grid,
        in_specs=[pl.BlockSpec((1, PBLK, D), lambda b, s: (b, s, 0))],
        out_specs=pl.BlockSpec((1, PBLK // SR, D), lambda b, s: (b, s, 0)),
        out_shape=jax.ShapeDtypeStruct((B, S // SR, D), jnp.int32),
        compiler_params=pltpu.CompilerParams(
            dimension_semantics=("parallel", "parallel")),
    )(mask8)


def _sc_body(x_hbm, m_hbm, o_hbm, xbuf, mbuf, obuf,
             is0, is1, is2, is3, os0, os1):
    wid = lax.axis_index("s") * NC + lax.axis_index("c")
    b = wid // ND
    d0 = (wid % ND) * DSTRIP
    isems = (is0, is1, is2, is3)
    osems = (os0, os1)

    def start_in(t, k):
        pltpu.async_copy(
            x_hbm.at[b, pl.ds(t * ST, ST), pl.ds(d0, DSTRIP)],
            xbuf.at[k], isems[k])
        pltpu.async_copy(
            m_hbm.at[b, pl.ds(t * (ST // SR), ST // SR), pl.ds(d0, DSTRIP)],
            mbuf.at[k], isems[k])

    def wait_in(k):
        pltpu.make_async_copy(
            x_hbm.at[0, pl.ds(0, ST), pl.ds(0, DSTRIP)],
            xbuf.at[k], isems[k]).wait()
        pltpu.make_async_copy(
            m_hbm.at[0, pl.ds(0, ST // SR), pl.ds(0, DSTRIP)],
            mbuf.at[k], isems[k]).wait()

    def start_out(t, k):
        pltpu.async_copy(
            obuf.at[k],
            o_hbm.at[b, pl.ds(t * ST, ST), pl.ds(d0, DSTRIP)],
            osems[k])

    def wait_out(k):
        pltpu.make_async_copy(
            obuf.at[k],
            o_hbm.at[0, pl.ds(0, ST), pl.ds(0, DSTRIP)],
            osems[k]).wait()

    for k in range(4):
        start_in(k, k)

    def rows(ki, ko, accs):
        def rowbody(r8, accs):
            wv = [mbuf[ki, r8, pl.ds(gg * 16, 16)] for gg in range(NG)]
            cur = list(accs)
            for g in range(SR):
                r = r8 * SR + g
                for gg in range(NG):
                    cond = (wv[gg] & (1 << g)) != 0
                    xg = xbuf[ki, r, pl.ds(gg * 16, 16)]
                    a = cur[gg] + jnp.where(cond, xg, 0.0)
                    obuf[ko, r, pl.ds(gg * 16, 16)] = a
                    cur[gg] = a
            return tuple(cur)
        return lax.fori_loop(0, ST // SR, rowbody, accs)

    accs = tuple(jnp.zeros((16,), jnp.float32) for _ in range(NG))

    def tile4(i4, accs):
        for j in range(4):
            t = i4 * 4 + j
            ko = j % 2

            wait_in(j)

            if j < 2:
                @pl.when(i4 >= 1)
                def _():
                    wait_out(ko)
            else:
                wait_out(ko)

            accs = rows(j, ko, accs)
            start_out(t, ko)

            @pl.when(t + 4 < NT)
            def _():
                start_in(t + 4, j)
        return accs

    lax.fori_loop(0, NT // 4, tile4, accs)
    wait_out(0)
    wait_out(1)


@functools.partial(
    pl.kernel,
    out_type=jax.ShapeDtypeStruct((B, S, D), jnp.float32),
    mesh=plsc.VectorSubcoreMesh(core_axis_name="c", subcore_axis_name="s"),
    scratch_types=[
        pltpu.VMEM((4, ST, DSTRIP), jnp.float32),
        pltpu.VMEM((4, ST // SR, DSTRIP), jnp.int32),
        pltpu.VMEM((2, ST, DSTRIP), jnp.float32),
        pltpu.SemaphoreType.DMA,
        pltpu.SemaphoreType.DMA,
        pltpu.SemaphoreType.DMA,
        pltpu.SemaphoreType.DMA,
        pltpu.SemaphoreType.DMA,
        pltpu.SemaphoreType.DMA,
    ],
)
def _sc_kernel(x_hbm, m_hbm, o_hbm, *rest):
    _sc_body(x_hbm, m_hbm, o_hbm, *rest)


def kernel(x, mask):
    return _sc_kernel(x, _pack(mask.view(jnp.uint8)))


# speedup vs baseline: 3.2088x; 1.0148x over previous
import functools
import jax
import jax.numpy as jnp
from jax import lax
from jax.experimental import pallas as pl
from jax.experimental.pallas import tpu as pltpu
from jax.experimental.pallas import tpu_sc as plsc

B, S, D = 2, 8192, 2048
NC, NSUB = 2, 16
NW = NC * NSUB
DSTRIP = 128
ND = D // DSTRIP
ST = 128
NT = S // ST
NG = DSTRIP // 16
SR = 8
PBLK = 1024


def _pack_body(m_ref, o_ref):
    m = m_ref[0].astype(jnp.int32)
    m3 = m.reshape(PBLK // SR, SR, D)
    sh = lax.broadcasted_iota(jnp.int32, (1, SR, 1), 1)
    o_ref[0] = (m3 << sh).sum(axis=1)


def _pack(mask8):
    grid = (B, S // PBLK)
    return pl.pallas_call(
        _pack_body,
        grid=grid,
        in_specs=[pl.BlockSpec((1, PBLK, D), lambda b, s: (b, s, 0))],
        out_specs=pl.BlockSpec((1, PBLK // SR, D), lambda b, s: (b, s, 0)),
        out_shape=jax.ShapeDtypeStruct((B, S // SR, D), jnp.int32),
        compiler_params=pltpu.CompilerParams(
            dimension_semantics=("parallel", "parallel")),
    )(mask8)


def _sc_body(x_hbm, m_hbm, o_hbm, xbuf, mbuf, obuf,
             is0, is1, is2, is3, os0, os1):
    wid = lax.axis_index("s") * NC + lax.axis_index("c")
    b = wid // ND
    d0 = (wid % ND) * DSTRIP
    isems = (is0, is1, is2, is3)
    osems = (os0, os1)

    def start_in(t, k):
        pltpu.async_copy(
            x_hbm.at[b, pl.ds(t * ST, ST), pl.ds(d0, DSTRIP)],
            xbuf.at[k], isems[k])
        pltpu.async_copy(
            m_hbm.at[b, pl.ds(t * (ST // SR), ST // SR), pl.ds(d0, DSTRIP)],
            mbuf.at[k], isems[k])

    def wait_in(k):
        pltpu.make_async_copy(
            x_hbm.at[0, pl.ds(0, ST), pl.ds(0, DSTRIP)],
            xbuf.at[k], isems[k]).wait()
        pltpu.make_async_copy(
            m_hbm.at[0, pl.ds(0, ST // SR), pl.ds(0, DSTRIP)],
            mbuf.at[k], isems[k]).wait()

    def start_out(t, k):
        pltpu.async_copy(
            obuf.at[k],
            o_hbm.at[b, pl.ds(t * ST, ST), pl.ds(d0, DSTRIP)],
            osems[k])

    def wait_out(k):
        pltpu.make_async_copy(
            obuf.at[k],
            o_hbm.at[0, pl.ds(0, ST), pl.ds(0, DSTRIP)],
            osems[k]).wait()

    for k in range(4):
        start_in(k, k)

    def rows(ki, ko, accs):
        def rowbody(r8, accs):
            wv = [mbuf[ki, r8, pl.ds(gg * 16, 16)] for gg in range(NG)]
            cur = list(accs)
            for g in range(SR):
                r = r8 * SR + g
                for gg in range(NG):
                    cond = (wv[gg] & (1 << g)) != 0
                    xg = xbuf[ki, r, pl.ds(gg * 16, 16)]
                    a = cur[gg] + jnp.where(cond, xg, 0.0)
                    obuf[ko, r, pl.ds(gg * 16, 16)] = a
                    cur[gg] = a
            return tuple(cur)
        return lax.fori_loop(0, ST // SR, rowbody, accs)

    accs = tuple(jnp.zeros((16,), jnp.float32) for _ in range(NG))

    def tile4(i4, accs):
        for j in range(4):
            t = i4 * 4 + j
            ko = j % 2

            wait_in(j)

            if j < 2:
                @pl.when(i4 >= 1)
                def _():
                    wait_out(ko)
            else:
                wait_out(ko)

            accs = rows(j, ko, accs)
            start_out(t, ko)

            @pl.when(t + 4 < NT)
            def _():
                start_in(t + 4, j)
        return accs

    lax.fori_loop(0, NT // 4, tile4, accs)
    wait_out(0)
    wait_out(1)


@functools.partial(
    pl.kernel,
    out_type=jax.ShapeDtypeStruct((B, S, D), jnp.float32),
    mesh=plsc.VectorSubcoreMesh(core_axis_name="c", subcore_axis_name="s"),
    scratch_types=[
        pltpu.VMEM((4, ST, DSTRIP), jnp.float32),
        pltpu.VMEM((4, ST // SR, DSTRIP), jnp.int32),
        pltpu.VMEM((2, ST, DSTRIP), jnp.float32),
        pltpu.SemaphoreType.DMA,
        pltpu.SemaphoreType.DMA,
        pltpu.SemaphoreType.DMA,
        pltpu.SemaphoreType.DMA,
        pltpu.SemaphoreType.DMA,
        pltpu.SemaphoreType.DMA,
    ],
)
def _sc_kernel(x_hbm, m_hbm, o_hbm, *rest):
    _sc_body(x_hbm, m_hbm, o_hbm, *rest)


def kernel(x, mask):
    return _sc_kernel(x, _pack(mask.view(jnp.uint8)))


# speedup vs baseline: 3.6250x; 1.1297x over previous
import functools
import jax
import jax.numpy as jnp
from jax import lax
from jax.experimental import pallas as pl
from jax.experimental.pallas import tpu as pltpu
from jax.experimental.pallas import tpu_sc as plsc

B, S, D = 2, 8192, 2048
NC, NSUB = 2, 16
NW = NC * NSUB
DSTRIP = 128
ND = D // DSTRIP
ST = 128
NT = S // ST
NG = DSTRIP // 16
SR = 8
PBLK = 1024


def _pack_body(m_ref, o_ref):
    ir = lax.broadcasted_iota(jnp.int32, (PBLK // SR, PBLK), 0)
    jr = lax.broadcasted_iota(jnp.int32, (PBLK // SR, PBLK), 1)
    diff = jr - SR * ir
    inr = (diff >= 0) & (diff < SR)
    lw = jnp.where(inr, jnp.int32(1) << (diff & (SR - 1)), 0
                   ).astype(jnp.bfloat16)
    m = m_ref[0].astype(jnp.bfloat16)
    o_ref[0] = jnp.dot(
        lw, m, preferred_element_type=jnp.float32).astype(jnp.int32)


def _pack(mask8):
    grid = (B, S // PBLK)
    return pl.pallas_call(
        _pack_body,
        grid=grid,
        in_specs=[pl.BlockSpec((1, PBLK, D), lambda b, s: (b, s, 0))],
        out_specs=pl.BlockSpec((1, PBLK // SR, D), lambda b, s: (b, s, 0)),
        out_shape=jax.ShapeDtypeStruct((B, S // SR, D), jnp.int32),
        compiler_params=pltpu.CompilerParams(
            dimension_semantics=("parallel", "parallel")),
    )(mask8)


def _sc_body(x_hbm, m_hbm, o_hbm, xbuf, mbuf, obuf,
             is0, is1, is2, is3, os0, os1):
    wid = lax.axis_index("s") * NC + lax.axis_index("c")
    b = wid // ND
    d0 = (wid % ND) * DSTRIP
    isems = (is0, is1, is2, is3)
    osems = (os0, os1)

    def start_in(t, k):
        pltpu.async_copy(
            x_hbm.at[b, pl.ds(t * ST, ST), pl.ds(d0, DSTRIP)],
            xbuf.at[k], isems[k])
        pltpu.async_copy(
            m_hbm.at[b, pl.ds(t * (ST // SR), ST // SR), pl.ds(d0, DSTRIP)],
            mbuf.at[k], isems[k])

    def wait_in(k):
        pltpu.make_async_copy(
            x_hbm.at[0, pl.ds(0, ST), pl.ds(0, DSTRIP)],
            xbuf.at[k], isems[k]).wait()
        pltpu.make_async_copy(
            m_hbm.at[0, pl.ds(0, ST // SR), pl.ds(0, DSTRIP)],
            mbuf.at[k], isems[k]).wait()

    def start_out(t, k):
        pltpu.async_copy(
            obuf.at[k],
            o_hbm.at[b, pl.ds(t * ST, ST), pl.ds(d0, DSTRIP)],
            osems[k])

    def wait_out(k):
        pltpu.make_async_copy(
            obuf.at[k],
            o_hbm.at[0, pl.ds(0, ST), pl.ds(0, DSTRIP)],
            osems[k]).wait()

    for k in range(4):
        start_in(k, k)

    def rows(ki, ko, accs):
        def rowbody(r8, accs):
            wv = [mbuf[ki, r8, pl.ds(gg * 16, 16)] for gg in range(NG)]
            cur = list(accs)
            for g in range(SR):
                r = r8 * SR + g
                for gg in range(NG):
                    cond = (wv[gg] & (1 << g)) != 0
                    xg = xbuf[ki, r, pl.ds(gg * 16, 16)]
                    a = cur[gg] + jnp.where(cond, xg, 0.0)
                    obuf[ko, r, pl.ds(gg * 16, 16)] = a
                    cur[gg] = a
            return tuple(cur)
        return lax.fori_loop(0, ST // SR, rowbody, accs)

    accs = tuple(jnp.zeros((16,), jnp.float32) for _ in range(NG))

    def tile4(i4, accs):
        for j in range(4):
            t = i4 * 4 + j
            ko = j % 2

            wait_in(j)

            if j < 2:
                @pl.when(i4 >= 1)
                def _():
                    wait_out(ko)
            else:
                wait_out(ko)

            accs = rows(j, ko, accs)
            start_out(t, ko)

            @pl.when(t + 4 < NT)
            def _():
                start_in(t + 4, j)
        return accs

    lax.fori_loop(0, NT // 4, tile4, accs)
    wait_out(0)
    wait_out(1)


@functools.partial(
    pl.kernel,
    out_type=jax.ShapeDtypeStruct((B, S, D), jnp.float32),
    mesh=plsc.VectorSubcoreMesh(core_axis_name="c", subcore_axis_name="s"),
    scratch_types=[
        pltpu.VMEM((4, ST, DSTRIP), jnp.float32),
        pltpu.VMEM((4, ST // SR, DSTRIP), jnp.int32),
        pltpu.VMEM((2, ST, DSTRIP), jnp.float32),
        pltpu.SemaphoreType.DMA,
        pltpu.SemaphoreType.DMA,
        pltpu.SemaphoreType.DMA,
        pltpu.SemaphoreType.DMA,
        pltpu.SemaphoreType.DMA,
        pltpu.SemaphoreType.DMA,
    ],
)
def _sc_kernel(x_hbm, m_hbm, o_hbm, *rest):
    _sc_body(x_hbm, m_hbm, o_hbm, *rest)


def kernel(x, mask):
    return _sc_kernel(x, _pack(mask.view(jnp.uint8)))
